# Initial kernel scaffold; baseline (speedup 1.0000x reference)
import jax
import jax.numpy as jnp
from jax.experimental import pallas as pl


def kernel(x, edge_index0, edge_index1, edge_index2, W_fc1, b_fc1, W1, b1, W2, b2, W3, b3, W4, b4, W5, b5, W_fc2, b_fc2):
    raise NotImplementedError("write your pallas kernel here")


# speedup vs baseline: 8.3231x; 8.3231x over previous
import functools

import jax
import jax.numpy as jnp
from jax import lax
from jax.experimental import pallas as pl
from jax.experimental.pallas import tpu as pltpu
from jax.experimental.pallas import tpu_sc as plsc

_N0 = 32768
_N1 = 8192
_N2 = 2048
_NX = 128
_NY = 256
_E0 = 131072
_E1 = 32768
_E2 = 8192
_NTOT = _N0 + _N1 + _N2
_ETOT = _E0 + _E1 + _E2

_NC = 2
_NS = 16
_NW = _NC * _NS
_CH = 128


def _sc_mesh():
    return plsc.VectorSubcoreMesh(
        core_axis_name="c", subcore_axis_name="s",
        num_cores=_NC, num_subcores=_NS)


@functools.lru_cache(maxsize=None)
def _deg_kernel():
    KD = _ETOT // (_CH * _NW)
    RPT = _NTOT // _NS
    mesh = _sc_mesh()

    @functools.partial(
        pl.kernel,
        mesh=mesh,
        out_type=jax.ShapeDtypeStruct((_NC, _NTOT, 16), jnp.float32),
        compiler_params=pltpu.CompilerParams(use_tc_tiling_on_sc=False),
        scratch_types=[
            pltpu.VMEM((KD, _CH), jnp.int32),
            pltpu.VMEM((_CH, 16), jnp.float32),
            pltpu.VMEM((_CH, 16), jnp.float32),
            pltpu.VMEM_SHARED((_NTOT, 16), jnp.float32),
        ],
    )
    def deg(dst_hbm, out_hbm, didx, ones_v, zbuf, acc):
        c = lax.axis_index("c")
        s = lax.axis_index("s")
        wid = c * _NS + s
        ones16 = jnp.ones((16,), jnp.float32)
        zeros16 = jnp.zeros((16,), jnp.float32)

        def fill(i, _):
            ones_v[i, :] = ones16
            zbuf[i, :] = zeros16
            return 0

        lax.fori_loop(0, _CH, fill, 0)

        base = s * RPT

        def zacc(j, _):
            pltpu.sync_copy(zbuf, acc.at[pl.ds(base + j * _CH, _CH)])
            return 0

        lax.fori_loop(0, RPT // _CH, zacc, 0)

        pltpu.sync_copy(dst_hbm.at[wid], didx)
        plsc.subcore_barrier()

        def chunk(j, _):
            pltpu.sync_copy(ones_v, acc.at[didx.at[j]], add=True)
            return 0

        lax.fori_loop(0, KD, chunk, 0)
        plsc.subcore_barrier()

        def wb(j, _):
            pltpu.sync_copy(
                acc.at[pl.ds(base + j * _CH, _CH)],
                out_hbm.at[c, pl.ds(base + j * _CH, _CH)],
            )
            return 0

        lax.fori_loop(0, RPT // _CH, wb, 0)

    return deg


@functools.lru_cache(maxsize=None)
def _seg_kernel(N, E):
    K = E // (_CH * _NW)
    RPT = N // _NS
    ZCH = min(RPT, _CH)
    mesh = _sc_mesh()

    @functools.partial(
        pl.kernel,
        mesh=mesh,
        out_type=jax.ShapeDtypeStruct((_NC, N, 32), jnp.float32),
        compiler_params=pltpu.CompilerParams(use_tc_tiling_on_sc=False),
        scratch_types=[
            pltpu.VMEM((K, _CH), jnp.int32),
            pltpu.VMEM((K, _CH), jnp.int32),
            pltpu.VMEM((_CH, 32), jnp.float32),
            pltpu.VMEM((ZCH, 32), jnp.float32),
            pltpu.VMEM_SHARED((N, 32), jnp.float32),
            pltpu.SemaphoreType.DMA,
        ],
    )
    def seg(y_hbm, src_hbm, dst_hbm, out_hbm, sidx, didx, rows, zbuf, acc, sem):
        c = lax.axis_index("c")
        s = lax.axis_index("s")
        wid = c * _NS + s
        zeros16 = jnp.zeros((16,), jnp.float32)

        def fill(i, _):
            zbuf[i, pl.ds(0, 16)] = zeros16
            zbuf[i, pl.ds(16, 16)] = zeros16
            return 0

        lax.fori_loop(0, ZCH, fill, 0)

        base = s * RPT

        def zacc(j, _):
            pltpu.sync_copy(zbuf, acc.at[pl.ds(base + j * ZCH, ZCH)])
            return 0

        lax.fori_loop(0, RPT // ZCH, zacc, 0)

        pltpu.sync_copy(src_hbm.at[wid], sidx)
        pltpu.sync_copy(dst_hbm.at[wid], didx)
        plsc.subcore_barrier()

        def chunk(j, _):
            pltpu.async_copy(y_hbm.at[sidx.at[j]], rows, sem).wait()
            pltpu.sync_copy(rows, acc.at[didx.at[j]], add=True)
            return 0

        lax.fori_loop(0, K, chunk, 0)
        plsc.subcore_barrier()

        def wb(j, _):
            pltpu.sync_copy(
                acc.at[pl.ds(base + j * ZCH, ZCH)],
                out_hbm.at[c, pl.ds(base + j * ZCH, ZCH)],
            )
            return 0

        lax.fori_loop(0, RPT // ZCH, wb, 0)

    return seg


def _dinv_body(deg_ref, dinv_ref):
    d = deg_ref[0] + deg_ref[1]
    dinv_ref[...] = lax.rsqrt(d[:, 0:1] + 1.0)


def _dinv_call(degp):
    blk = 2048
    grid = _NTOT // blk
    return pl.pallas_call(
        _dinv_body,
        grid=(grid,),
        in_specs=[pl.BlockSpec((_NC, blk, 16), lambda i: (0, i, 0))],
        out_specs=pl.BlockSpec((blk, 1), lambda i: (i, 0)),
        out_shape=jax.ShapeDtypeStruct((_NTOT, 1), jnp.float32),
    )(degp)


def _fc1prep_body(x_ref, wf_ref, bf_ref, w1_ref, dinv_ref, y_ref):
    h = jnp.dot(x_ref[...], wf_ref[...], preferred_element_type=jnp.float32)
    h = jnp.maximum(h + bf_ref[...], 0.0)
    y = jnp.dot(h, w1_ref[...], preferred_element_type=jnp.float32)
    y_ref[...] = y * dinv_ref[...]


def _fc1prep(x, wfT, bf, w1T, dinv):
    blk = 4096
    grid = _N0 // blk
    return pl.pallas_call(
        _fc1prep_body,
        grid=(grid,),
        in_specs=[
            pl.BlockSpec((blk, 4), lambda i: (i, 0)),
            pl.BlockSpec((4, 32), lambda i: (0, 0)),
            pl.BlockSpec((1, 32), lambda i: (0, 0)),
            pl.BlockSpec((32, 32), lambda i: (0, 0)),
            pl.BlockSpec((blk, 1), lambda i: (i, 0)),
        ],
        out_specs=pl.BlockSpec((blk, 32), lambda i: (i, 0)),
        out_shape=jax.ShapeDtypeStruct((_N0, 32), jnp.float32),
    )(x, wfT, bf, w1T, dinv)


def _prep_body(x_ref, w_ref, dinv_ref, y_ref):
    y = jnp.dot(x_ref[...], w_ref[...], preferred_element_type=jnp.float32)
    y_ref[...] = y * dinv_ref[...]


def _prep(x, wT, dinv):
    n = x.shape[0]
    blk = min(n, 4096)
    return pl.pallas_call(
        _prep_body,
        grid=(n // blk,),
        in_specs=[
            pl.BlockSpec((blk, 32), lambda i: (i, 0)),
            pl.BlockSpec((32, 32), lambda i: (0, 0)),
            pl.BlockSpec((blk, 1), lambda i: (i, 0)),
        ],
        out_specs=pl.BlockSpec((blk, 32), lambda i: (i, 0)),
        out_shape=jax.ShapeDtypeStruct((n, 32), jnp.float32),
    )(x, wT, dinv)


def _prep_add_body(a_ref, b_ref, w_ref, dinv_ref, y_ref):
    y = jnp.dot(a_ref[...] + b_ref[...], w_ref[...],
                preferred_element_type=jnp.float32)
    y_ref[...] = y * dinv_ref[...]


def _prep_add(a, b, wT, dinv):
    n = a.shape[0]
    blk = min(n, 4096)
    return pl.pallas_call(
        _prep_add_body,
        grid=(n // blk,),
        in_specs=[
            pl.BlockSpec((blk, 32), lambda i: (i, 0)),
            pl.BlockSpec((blk, 32), lambda i: (i, 0)),
            pl.BlockSpec((32, 32), lambda i: (0, 0)),
            pl.BlockSpec((blk, 1), lambda i: (i, 0)),
        ],
        out_specs=pl.BlockSpec((blk, 32), lambda i: (i, 0)),
        out_shape=jax.ShapeDtypeStruct((n, 32), jnp.float32),
    )(a, b, wT, dinv)


def _comb_body(acc_ref, y_ref, dinv_ref, b_ref, h_ref):
    a = acc_ref[0] + acc_ref[1] + y_ref[...]
    h_ref[...] = jnp.maximum(dinv_ref[...] * a + b_ref[...], 0.0)


def _comb(acc, y, dinv, b):
    n = y.shape[0]
    blk = min(n, 4096)
    return pl.pallas_call(
        _comb_body,
        grid=(n // blk,),
        in_specs=[
            pl.BlockSpec((_NC, blk, 32), lambda i: (0, i, 0)),
            pl.BlockSpec((blk, 32), lambda i: (i, 0)),
            pl.BlockSpec((blk, 1), lambda i: (i, 0)),
            pl.BlockSpec((1, 32), lambda i: (0, 0)),
        ],
        out_specs=pl.BlockSpec((blk, 32), lambda i: (i, 0)),
        out_shape=jax.ShapeDtypeStruct((n, 32), jnp.float32),
    )(acc, y, dinv, b)


def _comb_fc2_body(acc_ref, y_ref, dinv_ref, b_ref, w2_ref, b2_ref, o_ref):
    a = acc_ref[0] + acc_ref[1] + y_ref[...]
    h = jnp.maximum(dinv_ref[...] * a + b_ref[...], 0.0)
    o = jnp.dot(h, w2_ref[...], preferred_element_type=jnp.float32)
    o_ref[...] = o + b2_ref[...]


def _comb_fc2(acc, y, dinv, b, w2T, b2):
    blk = 4096
    return pl.pallas_call(
        _comb_fc2_body,
        grid=(_N0 // blk,),
        in_specs=[
            pl.BlockSpec((_NC, blk, 32), lambda i: (0, i, 0)),
            pl.BlockSpec((blk, 32), lambda i: (i, 0)),
            pl.BlockSpec((blk, 1), lambda i: (i, 0)),
            pl.BlockSpec((1, 32), lambda i: (0, 0)),
            pl.BlockSpec((32, 2), lambda i: (0, 0)),
            pl.BlockSpec((1, 2), lambda i: (0, 0)),
        ],
        out_specs=pl.BlockSpec((blk, 2), lambda i: (i, 0)),
        out_shape=jax.ShapeDtypeStruct((_N0, 2), jnp.float32),
    )(acc, y, dinv, b, w2T, b2)


def _down(h, nx, ny):
    return h.reshape(nx, ny, 32)[::2, ::2, :].reshape(-1, 32)


def _up(x, channels):
    x = x.T
    for _ in range(2):
        x = x.reshape(channels, -1)
        x = jnp.repeat(x, 2, axis=1)
        channels = channels * 2
    return x.T.reshape(-1, 32)


def kernel(x, edge_index0, edge_index1, edge_index2, W_fc1, b_fc1, W1, b1,
           W2, b2, W3, b3, W4, b4, W5, b5, W_fc2, b_fc2):
    src0 = edge_index0[0].reshape(_NW, -1, _CH)
    dst0 = edge_index0[1].reshape(_NW, -1, _CH)
    src1 = edge_index1[0].reshape(_NW, -1, _CH)
    dst1 = edge_index1[1].reshape(_NW, -1, _CH)
    src2 = edge_index2[0].reshape(_NW, -1, _CH)
    dst2 = edge_index2[1].reshape(_NW, -1, _CH)
    dst_all = jnp.concatenate(
        [edge_index0[1], edge_index1[1] + _N0, edge_index2[1] + (_N0 + _N1)]
    ).reshape(_NW, -1, _CH)

    degp = _deg_kernel()(dst_all)
    dinv_all = _dinv_call(degp)
    dinv0 = dinv_all[:_N0]
    dinv1 = dinv_all[_N0:_N0 + _N1]
    dinv2 = dinv_all[_N0 + _N1:]

    b1r = b1.reshape(1, 32)
    b2r = b2.reshape(1, 32)
    b3r = b3.reshape(1, 32)
    b4r = b4.reshape(1, 32)
    b5r = b5.reshape(1, 32)

    y1 = _fc1prep(x, W_fc1.T, b_fc1.reshape(1, 32), W1.T, dinv0)
    acc1 = _seg_kernel(_N0, _E0)(y1, src0, dst0)
    hb = _comb(acc1, y1, dinv0, b1r)

    x1 = _down(hb, _NX, _NY)
    y2 = _prep(x1, W2.T, dinv1)
    acc2 = _seg_kernel(_N1, _E1)(y2, src1, dst1)
    x1c = _comb(acc2, y2, dinv1, b2r)

    x2 = _down(x1c, _NX // 2, _NY // 2)
    y3 = _prep(x2, W3.T, dinv2)
    acc3 = _seg_kernel(_N2, _E2)(y3, src2, dst2)
    x2c = _comb(acc3, y3, dinv2, b3r)

    x2u = _up(x2c, 32)
    y4 = _prep_add(x1c, x2u, W4.T, dinv1)
    acc4 = _seg_kernel(_N1, _E1)(y4, src1, dst1)
    x1uc = _comb(acc4, y4, dinv1, b4r)

    xu = _up(x1uc, 32)
    y5 = _prep_add(hb, xu, W5.T, dinv0)
    acc5 = _seg_kernel(_N0, _E0)(y5, src0, dst0)
    return _comb_fc2(acc5, y5, dinv0, b5r, W_fc2.T, b_fc2.reshape(1, 2))
